# baseline (device time: 128992 ns/iter reference)
import math

import jax
import jax.numpy as jnp
from jax import lax
from jax.experimental import pallas as pl
from jax.experimental.pallas import tpu as pltpu

N_DEV = 16
B = 2
SQ = 128
D = 512
HQ = 4
DH = 64
DQK = HQ * DH
ROWS = B * SQ


def kernel(x, Wq, Wk, Wv, Wo):
    def body(x_ref, wq_ref, wk_ref, wv_ref, wo_ref, out_ref,
             kv_ref, ctx_ref, send_sems, recv_sems):
        my = lax.axis_index("i")
        left = lax.rem(my + N_DEV - 1, N_DEV)
        right = lax.rem(my + 1, N_DEV)

        x2 = x_ref[...].reshape(ROWS, D)
        q = jnp.dot(x2, wq_ref[...], preferred_element_type=jnp.float32)
        k = jnp.dot(x2, wk_ref[...], preferred_element_type=jnp.float32)
        v = jnp.dot(x2, wv_ref[...], preferred_element_type=jnp.float32)

        row = lax.broadcasted_iota(jnp.int32, (ROWS, DQK), 0)
        col = lax.broadcasted_iota(jnp.int32, (ROWS, DQK), 1)
        pos = (lax.rem(row, SQ) + my * SQ).astype(jnp.float32)
        expo = (((lax.rem(col, DH) // 2) * 2).astype(jnp.float32)) / DH
        inv = jnp.exp(-expo * math.log(10000.0))
        angle = pos * inv
        cosv = jnp.cos(angle)
        sinv = jnp.sin(angle)

        jj = lax.broadcasted_iota(jnp.int32, (DQK, DQK), 0)
        cc = lax.broadcasted_iota(jnp.int32, (DQK, DQK), 1)
        rot = jnp.where((lax.rem(cc, 2) == 0) & (jj == cc + 1), -1.0,
                        jnp.where((lax.rem(cc, 2) == 1) & (jj == cc - 1),
                                  1.0, 0.0)).astype(jnp.float32)

        q = q * cosv + jnp.dot(q, rot, preferred_element_type=jnp.float32) * sinv
        k = k * cosv + jnp.dot(k, rot, preferred_element_type=jnp.float32) * sinv

        kv_ref[0] = jnp.concatenate([k, v], axis=1)

        barrier = pltpu.get_barrier_semaphore()
        for nbr in (left, right):
            pl.semaphore_signal(barrier, inc=1, device_id=(nbr,),
                                device_id_type=pl.DeviceIdType.MESH)
        pl.semaphore_wait(barrier, 2)

        for h in range(N_DEV - 1):
            rdma = pltpu.make_async_remote_copy(
                src_ref=kv_ref.at[h],
                dst_ref=kv_ref.at[h + 1],
                send_sem=send_sems.at[h],
                recv_sem=recv_sems.at[h],
                device_id=(right,),
                device_id_type=pl.DeviceIdType.MESH,
            )
            rdma.start()
            rdma.wait()

        for b in range(B):
            r0 = b * SQ
            for hh in range(HQ):
                c0 = hh * DH
                qbh = q[r0:r0 + SQ, c0:c0 + DH]
                kf = kv_ref[:, r0:r0 + SQ, c0:c0 + DH].reshape(N_DEV * SQ, DH)
                vf = kv_ref[:, r0:r0 + SQ,
                            DQK + c0:DQK + c0 + DH].reshape(N_DEV * SQ, DH)
                s = lax.dot_general(
                    qbh, kf, (((1,), (1,)), ((), ())),
                    preferred_element_type=jnp.float32) * 0.125
                m = jnp.max(s, axis=1, keepdims=True)
                w = jnp.exp(s - m)
                w = w / jnp.sum(w, axis=1, keepdims=True)
                ctx_ref[r0:r0 + SQ, c0:c0 + DH] = jnp.dot(
                    w, vf, preferred_element_type=jnp.float32)

        out2 = jnp.dot(ctx_ref[...], wo_ref[...],
                       preferred_element_type=jnp.float32)
        out_ref[...] = out2.reshape(B, SQ, D)

    return pl.pallas_call(
        body,
        out_shape=jax.ShapeDtypeStruct((B, SQ, D), jnp.float32),
        in_specs=[pl.BlockSpec(memory_space=pltpu.VMEM)] * 5,
        out_specs=pl.BlockSpec(memory_space=pltpu.VMEM),
        scratch_shapes=[
            pltpu.VMEM((N_DEV, ROWS, 2 * DQK), jnp.float32),
            pltpu.VMEM((ROWS, DQK), jnp.float32),
            pltpu.SemaphoreType.DMA((N_DEV - 1,)),
            pltpu.SemaphoreType.DMA((N_DEV - 1,)),
        ],
        compiler_params=pltpu.CompilerParams(collective_id=0),
    )(x, Wq, Wk, Wv, Wo)


# device time: 72485 ns/iter; 1.7796x vs baseline; 1.7796x over previous
import math

import jax
import jax.numpy as jnp
from jax import lax
from jax.experimental import pallas as pl
from jax.experimental.pallas import tpu as pltpu

N_DEV = 16
B = 2
SQ = 128
D = 512
HQ = 4
DH = 64
DQK = HQ * DH
ROWS = B * SQ
R_HOPS = 8
L_HOPS = 7


def kernel(x, Wq, Wk, Wv, Wo):
    def body(x_ref, wq_ref, wk_ref, wv_ref, wo_ref, out_ref,
             kv_ref, send_r, recv_r, send_l, recv_l):
        my = lax.axis_index("i")
        left = lax.rem(my + N_DEV - 1, N_DEV)
        right = lax.rem(my + 1, N_DEV)

        x2 = x_ref[...].reshape(ROWS, D)
        q = jnp.dot(x2, wq_ref[...], preferred_element_type=jnp.float32)
        k = jnp.dot(x2, wk_ref[...], preferred_element_type=jnp.float32)
        v = jnp.dot(x2, wv_ref[...], preferred_element_type=jnp.float32)

        row = lax.broadcasted_iota(jnp.int32, (ROWS, DQK), 0)
        col = lax.broadcasted_iota(jnp.int32, (ROWS, DQK), 1)
        pos = (lax.rem(row, SQ) + my * SQ).astype(jnp.float32)
        expo = (((lax.rem(col, DH) // 2) * 2).astype(jnp.float32)) / DH
        inv = jnp.exp(-expo * math.log(10000.0))
        angle = pos * inv
        cosv = jnp.cos(angle)
        sinv = jnp.sin(angle)

        jj = lax.broadcasted_iota(jnp.int32, (DQK, DQK), 0)
        cc = lax.broadcasted_iota(jnp.int32, (DQK, DQK), 1)
        rot = jnp.where((lax.rem(cc, 2) == 0) & (jj == cc + 1), -1.0,
                        jnp.where((lax.rem(cc, 2) == 1) & (jj == cc - 1),
                                  1.0, 0.0)).astype(jnp.float32)

        q = q * cosv + jnp.dot(q, rot, preferred_element_type=jnp.float32) * sinv
        k = k * cosv + jnp.dot(k, rot, preferred_element_type=jnp.float32) * sinv

        kv_ref[0] = jnp.concatenate([k, v], axis=1)

        barrier = pltpu.get_barrier_semaphore()
        for nbr in (left, right):
            pl.semaphore_signal(barrier, inc=1, device_id=(nbr,),
                                device_id_type=pl.DeviceIdType.MESH)
        pl.semaphore_wait(barrier, 2)

        lsum = [jnp.zeros((SQ, 1), jnp.float32) for _ in range(B * HQ)]
        acc = [jnp.zeros((SQ, DH), jnp.float32) for _ in range(B * HQ)]

        def process(slot):
            for b in range(B):
                r0 = b * SQ
                for hh in range(HQ):
                    c0 = hh * DH
                    i = b * HQ + hh
                    qbh = q[r0:r0 + SQ, c0:c0 + DH]
                    kc = kv_ref[slot, r0:r0 + SQ, c0:c0 + DH]
                    vc = kv_ref[slot, r0:r0 + SQ, DQK + c0:DQK + c0 + DH]
                    s = lax.dot_general(
                        qbh, kc, (((1,), (1,)), ((), ())),
                        preferred_element_type=jnp.float32) * 0.125
                    p = jnp.exp(s)
                    lsum[i] = lsum[i] + jnp.sum(p, axis=1, keepdims=True)
                    acc[i] = acc[i] + jnp.dot(
                        p, vc, preferred_element_type=jnp.float32)

        def start(src_slot, dst_slot, sems_s, sems_r, h, dst_dev):
            rdma = pltpu.make_async_remote_copy(
                src_ref=kv_ref.at[src_slot],
                dst_ref=kv_ref.at[dst_slot],
                send_sem=sems_s.at[h],
                recv_sem=sems_r.at[h],
                device_id=(dst_dev,),
                device_id_type=pl.DeviceIdType.MESH,
            )
            rdma.start()
            return rdma

        rdmas = [start(0, 1, send_r, recv_r, 0, right),
                 start(0, 15, send_l, recv_l, 0, left)]
        process(0)

        for h in range(1, R_HOPS + 1):
            rdmas[2 * (h - 1)].wait_recv()
            if h < R_HOPS:
                rdmas.append(start(h, h + 1, send_r, recv_r, h, right))
            if h <= L_HOPS:
                rdmas[2 * (h - 1) + 1].wait_recv()
                if h < L_HOPS:
                    rdmas.append(
                        start(16 - h, 15 - h, send_l, recv_l, h, left))
            process(h)
            if h <= L_HOPS:
                process(16 - h)

        ctx = jnp.concatenate(
            [jnp.concatenate([acc[b * HQ + hh] / lsum[b * HQ + hh]
                              for hh in range(HQ)], axis=1)
             for b in range(B)], axis=0)
        out2 = jnp.dot(ctx, wo_ref[...], preferred_element_type=jnp.float32)
        out_ref[...] = out2.reshape(B, SQ, D)

        for r in rdmas:
            r.wait_send()

    return pl.pallas_call(
        body,
        out_shape=jax.ShapeDtypeStruct((B, SQ, D), jnp.float32),
        in_specs=[pl.BlockSpec(memory_space=pltpu.VMEM)] * 5,
        out_specs=pl.BlockSpec(memory_space=pltpu.VMEM),
        scratch_shapes=[
            pltpu.VMEM((N_DEV, ROWS, 2 * DQK), jnp.float32),
            pltpu.SemaphoreType.DMA((R_HOPS,)),
            pltpu.SemaphoreType.DMA((R_HOPS,)),
            pltpu.SemaphoreType.DMA((L_HOPS,)),
            pltpu.SemaphoreType.DMA((L_HOPS,)),
        ],
        compiler_params=pltpu.CompilerParams(collective_id=0),
    )(x, Wq, Wk, Wv, Wo)


# device time: 50238 ns/iter; 2.5676x vs baseline; 1.4428x over previous
import math

import jax
import jax.numpy as jnp
from jax import lax
from jax.experimental import pallas as pl
from jax.experimental.pallas import tpu as pltpu

N_DEV = 16
B = 2
SQ = 128
D = 512
HQ = 4
DH = 64
DQK = HQ * DH
ROWS = B * SQ
R_HOPS = 8
L_HOPS = 7


def kernel(x, Wq, Wk, Wv, Wo):
    def body(x_ref, wq_ref, wk_ref, wv_ref, wo_ref, out_ref,
             kv_ref, send_r, recv_r, send_l, recv_l):
        my = lax.axis_index("i")
        left = lax.rem(my + N_DEV - 1, N_DEV)
        right = lax.rem(my + 1, N_DEV)

        x2 = x_ref[...].reshape(ROWS, D)
        q = jnp.dot(x2, wq_ref[...], preferred_element_type=jnp.float32)
        k = jnp.dot(x2, wk_ref[...], preferred_element_type=jnp.float32)
        v = jnp.dot(x2, wv_ref[...], preferred_element_type=jnp.float32)

        row = lax.broadcasted_iota(jnp.int32, (ROWS, DQK), 0)
        col = lax.broadcasted_iota(jnp.int32, (ROWS, DQK), 1)
        pos = (lax.rem(row, SQ) + my * SQ).astype(jnp.float32)
        expo = (((lax.rem(col, DH) // 2) * 2).astype(jnp.float32)) / DH
        inv = jnp.exp(-expo * math.log(10000.0))
        angle = pos * inv
        cosv = jnp.cos(angle)
        sinv = jnp.sin(angle)

        jj = lax.broadcasted_iota(jnp.int32, (DQK, DQK), 0)
        cc = lax.broadcasted_iota(jnp.int32, (DQK, DQK), 1)
        rot = jnp.where((lax.rem(cc, 2) == 0) & (jj == cc + 1), -1.0,
                        jnp.where((lax.rem(cc, 2) == 1) & (jj == cc - 1),
                                  1.0, 0.0)).astype(jnp.float32)

        q = q * cosv + jnp.dot(q, rot, preferred_element_type=jnp.float32) * sinv
        k = k * cosv + jnp.dot(k, rot, preferred_element_type=jnp.float32) * sinv

        kv_ref[0] = jnp.concatenate([k, v], axis=1).astype(jnp.bfloat16)

        barrier = pltpu.get_barrier_semaphore()
        for nbr in (left, right):
            pl.semaphore_signal(barrier, inc=1, device_id=(nbr,),
                                device_id_type=pl.DeviceIdType.MESH)
        pl.semaphore_wait(barrier, 2)

        lsum = [jnp.zeros((SQ, 1), jnp.float32) for _ in range(B * HQ)]
        acc = [jnp.zeros((SQ, DH), jnp.float32) for _ in range(B * HQ)]

        def process(slot):
            for b in range(B):
                r0 = b * SQ
                for hh in range(HQ):
                    c0 = hh * DH
                    i = b * HQ + hh
                    qbh = q[r0:r0 + SQ, c0:c0 + DH]
                    kc = kv_ref[slot, r0:r0 + SQ,
                                c0:c0 + DH].astype(jnp.float32)
                    vc = kv_ref[slot, r0:r0 + SQ,
                                DQK + c0:DQK + c0 + DH].astype(jnp.float32)
                    s = lax.dot_general(
                        qbh, kc, (((1,), (1,)), ((), ())),
                        preferred_element_type=jnp.float32) * 0.125
                    p = jnp.exp(s)
                    lsum[i] = lsum[i] + jnp.sum(p, axis=1, keepdims=True)
                    acc[i] = acc[i] + jnp.dot(
                        p, vc, preferred_element_type=jnp.float32)

        def start(src_slot, dst_slot, sems_s, sems_r, h, dst_dev):
            rdma = pltpu.make_async_remote_copy(
                src_ref=kv_ref.at[src_slot],
                dst_ref=kv_ref.at[dst_slot],
                send_sem=sems_s.at[h],
                recv_sem=sems_r.at[h],
                device_id=(dst_dev,),
                device_id_type=pl.DeviceIdType.MESH,
            )
            rdma.start()
            return rdma

        rdmas = [start(0, 1, send_r, recv_r, 0, right),
                 start(0, 15, send_l, recv_l, 0, left)]
        process(0)

        for h in range(1, R_HOPS + 1):
            rdmas[2 * (h - 1)].wait_recv()
            if h < R_HOPS:
                rdmas.append(start(h, h + 1, send_r, recv_r, h, right))
            if h <= L_HOPS:
                rdmas[2 * (h - 1) + 1].wait_recv()
                if h < L_HOPS:
                    rdmas.append(
                        start(16 - h, 15 - h, send_l, recv_l, h, left))
            process(h)
            if h <= L_HOPS:
                process(16 - h)

        ctx = jnp.concatenate(
            [jnp.concatenate([acc[b * HQ + hh] / lsum[b * HQ + hh]
                              for hh in range(HQ)], axis=1)
             for b in range(B)], axis=0)
        out2 = jnp.dot(ctx, wo_ref[...], preferred_element_type=jnp.float32)
        out_ref[...] = out2.reshape(B, SQ, D)

        for r in rdmas:
            r.wait_send()

    return pl.pallas_call(
        body,
        out_shape=jax.ShapeDtypeStruct((B, SQ, D), jnp.float32),
        in_specs=[pl.BlockSpec(memory_space=pltpu.VMEM)] * 5,
        out_specs=pl.BlockSpec(memory_space=pltpu.VMEM),
        scratch_shapes=[
            pltpu.VMEM((N_DEV, ROWS, 2 * DQK), jnp.bfloat16),
            pltpu.SemaphoreType.DMA((R_HOPS,)),
            pltpu.SemaphoreType.DMA((R_HOPS,)),
            pltpu.SemaphoreType.DMA((L_HOPS,)),
            pltpu.SemaphoreType.DMA((L_HOPS,)),
        ],
        compiler_params=pltpu.CompilerParams(collective_id=0),
    )(x, Wq, Wk, Wv, Wo)


# device time: 46730 ns/iter; 2.7604x vs baseline; 1.0751x over previous
import math

import jax
import jax.numpy as jnp
from jax import lax
from jax.experimental import pallas as pl
from jax.experimental.pallas import tpu as pltpu

N_DEV = 16
B = 2
SQ = 128
D = 512
HQ = 4
DH = 64
DQK = HQ * DH
ROWS = B * SQ
R_HOPS = 8
L_HOPS = 7


def kernel(x, Wq, Wk, Wv, Wo):
    def body(x_ref, wq_ref, wk_ref, wv_ref, wo_ref, out_ref,
             kv_ref, send_r, recv_r, send_l, recv_l):
        my = lax.axis_index("i")
        left = lax.rem(my + N_DEV - 1, N_DEV)
        right = lax.rem(my + 1, N_DEV)

        x2 = x_ref[...].reshape(ROWS, D)
        q = jnp.dot(x2, wq_ref[...], preferred_element_type=jnp.float32)
        k = jnp.dot(x2, wk_ref[...], preferred_element_type=jnp.float32)
        v = jnp.dot(x2, wv_ref[...], preferred_element_type=jnp.float32)

        row = lax.broadcasted_iota(jnp.int32, (ROWS, DQK), 0)
        col = lax.broadcasted_iota(jnp.int32, (ROWS, DQK), 1)
        pos = (lax.rem(row, SQ) + my * SQ).astype(jnp.float32)
        expo = (((lax.rem(col, DH) // 2) * 2).astype(jnp.float32)) / DH
        inv = jnp.exp(-expo * math.log(10000.0))
        angle = pos * inv
        cosv = jnp.cos(angle)
        sinv = jnp.sin(angle)

        jj = lax.broadcasted_iota(jnp.int32, (DQK, DQK), 0)
        cc = lax.broadcasted_iota(jnp.int32, (DQK, DQK), 1)
        rot = jnp.where((lax.rem(cc, 2) == 0) & (jj == cc + 1), -1.0,
                        jnp.where((lax.rem(cc, 2) == 1) & (jj == cc - 1),
                                  1.0, 0.0)).astype(jnp.float32)

        q = q * cosv + jnp.dot(q, rot, preferred_element_type=jnp.float32) * sinv
        k = k * cosv + jnp.dot(k, rot, preferred_element_type=jnp.float32) * sinv

        kv_ref[0] = jnp.concatenate([k, v], axis=1).astype(jnp.bfloat16)

        barrier = pltpu.get_barrier_semaphore()
        for nbr in (left, right):
            pl.semaphore_signal(barrier, inc=1, device_id=(nbr,),
                                device_id_type=pl.DeviceIdType.MESH)
        pl.semaphore_wait(barrier, 2)

        lsum = [jnp.zeros((SQ, 1), jnp.float32) for _ in range(B * HQ)]
        acc = [jnp.zeros((SQ, DH), jnp.float32) for _ in range(B * HQ)]

        def process_b(slot, b):
            r0 = b * SQ
            for hh in range(HQ):
                c0 = hh * DH
                i = b * HQ + hh
                qbh = q[r0:r0 + SQ, c0:c0 + DH]
                kc = kv_ref[slot, r0:r0 + SQ,
                            c0:c0 + DH].astype(jnp.float32)
                vc = kv_ref[slot, r0:r0 + SQ,
                            DQK + c0:DQK + c0 + DH].astype(jnp.float32)
                s = lax.dot_general(
                    qbh, kc, (((1,), (1,)), ((), ())),
                    preferred_element_type=jnp.float32) * 0.125
                p = jnp.exp(s)
                lsum[i] = lsum[i] + jnp.sum(p, axis=1, keepdims=True)
                acc[i] = acc[i] + jnp.dot(
                    p, vc, preferred_element_type=jnp.float32)

        def start(src_slot, dst_slot, sems_s, sems_r, h, half, dst_dev):
            rows = slice(half * SQ, (half + 1) * SQ)
            rdma = pltpu.make_async_remote_copy(
                src_ref=kv_ref.at[src_slot, rows],
                dst_ref=kv_ref.at[dst_slot, rows],
                send_sem=sems_s.at[h, half],
                recv_sem=sems_r.at[h, half],
                device_id=(dst_dev,),
                device_id_type=pl.DeviceIdType.MESH,
            )
            rdma.start()
            return rdma

        dR = [[None, None] for _ in range(R_HOPS)]
        dL = [[None, None] for _ in range(L_HOPS)]

        for half in (0, 1):
            dR[0][half] = start(0, 1, send_r, recv_r, 0, half, right)
            dL[0][half] = start(0, 15, send_l, recv_l, 0, half, left)
        process_b(0, 0)
        process_b(0, 1)

        for h in range(1, R_HOPS + 1):
            for half in (0, 1):
                dR[h - 1][half].wait_recv()
                if h < R_HOPS:
                    dR[h][half] = start(
                        h, h + 1, send_r, recv_r, h, half, right)
                if h <= L_HOPS:
                    dL[h - 1][half].wait_recv()
                    if h < L_HOPS:
                        dL[h][half] = start(
                            16 - h, 15 - h, send_l, recv_l, h, half, left)
                process_b(h, half)
                if h <= L_HOPS:
                    process_b(16 - h, half)

        ctx = jnp.concatenate(
            [jnp.concatenate([acc[b * HQ + hh] / lsum[b * HQ + hh]
                              for hh in range(HQ)], axis=1)
             for b in range(B)], axis=0)
        out2 = jnp.dot(ctx, wo_ref[...], preferred_element_type=jnp.float32)
        out_ref[...] = out2.reshape(B, SQ, D)

        for ds in dR + dL:
            for r in ds:
                r.wait_send()

    return pl.pallas_call(
        body,
        out_shape=jax.ShapeDtypeStruct((B, SQ, D), jnp.float32),
        in_specs=[pl.BlockSpec(memory_space=pltpu.VMEM)] * 5,
        out_specs=pl.BlockSpec(memory_space=pltpu.VMEM),
        scratch_shapes=[
            pltpu.VMEM((N_DEV, ROWS, 2 * DQK), jnp.bfloat16),
            pltpu.SemaphoreType.DMA((R_HOPS, 2)),
            pltpu.SemaphoreType.DMA((R_HOPS, 2)),
            pltpu.SemaphoreType.DMA((L_HOPS, 2)),
            pltpu.SemaphoreType.DMA((L_HOPS, 2)),
        ],
        compiler_params=pltpu.CompilerParams(collective_id=0),
    )(x, Wq, Wk, Wv, Wo)


# device time: 46646 ns/iter; 2.7653x vs baseline; 1.0018x over previous
import math

import jax
import jax.numpy as jnp
from jax import lax
from jax.experimental import pallas as pl
from jax.experimental.pallas import tpu as pltpu

N_DEV = 16
B = 2
SQ = 128
D = 512
HQ = 4
DH = 64
DQK = HQ * DH
ROWS = B * SQ
R_HOPS = 8
L_HOPS = 7


def kernel(x, Wq, Wk, Wv, Wo):
    def body(x_ref, wq_ref, wk_ref, wv_ref, wo_ref, out_ref,
             kv_ref, send_r, recv_r, send_l, recv_l):
        my = lax.axis_index("i")
        left = lax.rem(my + N_DEV - 1, N_DEV)
        right = lax.rem(my + 1, N_DEV)

        x2 = x_ref[...].reshape(ROWS, D)
        q = jnp.dot(x2, wq_ref[...], preferred_element_type=jnp.float32)
        k = jnp.dot(x2, wk_ref[...], preferred_element_type=jnp.float32)
        v = jnp.dot(x2, wv_ref[...], preferred_element_type=jnp.float32)

        row = lax.broadcasted_iota(jnp.int32, (ROWS, DQK), 0)
        col = lax.broadcasted_iota(jnp.int32, (ROWS, DQK), 1)
        pos = (lax.rem(row, SQ) + my * SQ).astype(jnp.float32)
        expo = (((lax.rem(col, DH) // 2) * 2).astype(jnp.float32)) / DH
        inv = jnp.exp(-expo * math.log(10000.0))
        angle = pos * inv
        cosv = jnp.cos(angle)
        sinv = jnp.sin(angle)

        jj = lax.broadcasted_iota(jnp.int32, (DQK, DQK), 0)
        cc = lax.broadcasted_iota(jnp.int32, (DQK, DQK), 1)
        rot = jnp.where((lax.rem(cc, 2) == 0) & (jj == cc + 1), -1.0,
                        jnp.where((lax.rem(cc, 2) == 1) & (jj == cc - 1),
                                  1.0, 0.0)).astype(jnp.float32)

        q = q * cosv + jnp.dot(q, rot, preferred_element_type=jnp.float32) * sinv
        k = k * cosv + jnp.dot(k, rot, preferred_element_type=jnp.float32) * sinv

        kv_ref[0] = jnp.concatenate([k, v], axis=1).astype(jnp.bfloat16)

        barrier = pltpu.get_barrier_semaphore()
        for nbr in (left, right):
            pl.semaphore_signal(barrier, inc=1, device_id=(nbr,),
                                device_id_type=pl.DeviceIdType.MESH)
        pl.semaphore_wait(barrier, 2)

        lsum = [jnp.zeros((SQ, 1), jnp.float32) for _ in range(B * HQ)]
        acc = [jnp.zeros((SQ, DH), jnp.float32) for _ in range(B * HQ)]

        q_bf = (q * 0.125).astype(jnp.bfloat16)

        def process_b(slot, b):
            r0 = b * SQ
            for hh in range(HQ):
                c0 = hh * DH
                i = b * HQ + hh
                qbh = q_bf[r0:r0 + SQ, c0:c0 + DH]
                kc = kv_ref[slot, r0:r0 + SQ, c0:c0 + DH]
                vc = kv_ref[slot, r0:r0 + SQ, DQK + c0:DQK + c0 + DH]
                s = lax.dot_general(
                    qbh, kc, (((1,), (1,)), ((), ())),
                    preferred_element_type=jnp.float32)
                p = jnp.exp(s)
                lsum[i] = lsum[i] + jnp.sum(p, axis=1, keepdims=True)
                acc[i] = acc[i] + jnp.dot(
                    p.astype(jnp.bfloat16), vc,
                    preferred_element_type=jnp.float32)

        def start(src_slot, dst_slot, sems_s, sems_r, h, half, dst_dev):
            rows = slice(half * SQ, (half + 1) * SQ)
            rdma = pltpu.make_async_remote_copy(
                src_ref=kv_ref.at[src_slot, rows],
                dst_ref=kv_ref.at[dst_slot, rows],
                send_sem=sems_s.at[h, half],
                recv_sem=sems_r.at[h, half],
                device_id=(dst_dev,),
                device_id_type=pl.DeviceIdType.MESH,
            )
            rdma.start()
            return rdma

        dR = [[None, None] for _ in range(R_HOPS)]
        dL = [[None, None] for _ in range(L_HOPS)]

        for half in (0, 1):
            dR[0][half] = start(0, 1, send_r, recv_r, 0, half, right)
            dL[0][half] = start(0, 15, send_l, recv_l, 0, half, left)
        process_b(0, 0)
        process_b(0, 1)

        for h in range(1, R_HOPS + 1):
            for half in (0, 1):
                dR[h - 1][half].wait_recv()
                if h < R_HOPS:
                    dR[h][half] = start(
                        h, h + 1, send_r, recv_r, h, half, right)
                if h <= L_HOPS:
                    dL[h - 1][half].wait_recv()
                    if h < L_HOPS:
                        dL[h][half] = start(
                            16 - h, 15 - h, send_l, recv_l, h, half, left)
            for half in (0, 1):
                process_b(h, half)
                if h <= L_HOPS:
                    process_b(16 - h, half)

        ctx = jnp.concatenate(
            [jnp.concatenate([acc[b * HQ + hh] / lsum[b * HQ + hh]
                              for hh in range(HQ)], axis=1)
             for b in range(B)], axis=0)
        out2 = jnp.dot(ctx, wo_ref[...], preferred_element_type=jnp.float32)
        out_ref[...] = out2.reshape(B, SQ, D)

        for ds in dR + dL:
            for r in ds:
                r.wait_send()

    return pl.pallas_call(
        body,
        out_shape=jax.ShapeDtypeStruct((B, SQ, D), jnp.float32),
        in_specs=[pl.BlockSpec(memory_space=pltpu.VMEM)] * 5,
        out_specs=pl.BlockSpec(memory_space=pltpu.VMEM),
        scratch_shapes=[
            pltpu.VMEM((N_DEV, ROWS, 2 * DQK), jnp.bfloat16),
            pltpu.SemaphoreType.DMA((R_HOPS, 2)),
            pltpu.SemaphoreType.DMA((R_HOPS, 2)),
            pltpu.SemaphoreType.DMA((L_HOPS, 2)),
            pltpu.SemaphoreType.DMA((L_HOPS, 2)),
        ],
        compiler_params=pltpu.CompilerParams(collective_id=0),
    )(x, Wq, Wk, Wv, Wo)


# device time: 39220 ns/iter; 3.2889x vs baseline; 1.1893x over previous
import math

import jax
import jax.numpy as jnp
from jax import lax
from jax.experimental import pallas as pl
from jax.experimental.pallas import tpu as pltpu

N_DEV = 16
B = 2
SQ = 128
D = 512
HQ = 4
DH = 64
DQK = HQ * DH
ROWS = B * SQ
R_HOPS = 8
L_HOPS = 7


def kernel(x, Wq, Wk, Wv, Wo):
    def body(x_ref, wq_ref, wk_ref, wv_ref, wo_ref, out_ref,
             kv_ref, send_r, recv_r, send_l, recv_l):
        my = lax.axis_index("i")

        def ring_pos(m):
            z, c = m // 4, lax.rem(m, 4)
            return c * 4 + jnp.where(lax.rem(c, 2) == 0, z, 3 - z)

        def ring_to_logical(r):
            c, w = r // 4, lax.rem(r, 4)
            z = jnp.where(lax.rem(c, 2) == 0, w, 3 - w)
            return 4 * z + c

        my_r = ring_pos(my)
        right = ring_to_logical(lax.rem(my_r + 1, N_DEV))
        left = ring_to_logical(lax.rem(my_r + N_DEV - 1, N_DEV))

        x2 = x_ref[...].reshape(ROWS, D)
        q = jnp.dot(x2, wq_ref[...], preferred_element_type=jnp.float32)
        k = jnp.dot(x2, wk_ref[...], preferred_element_type=jnp.float32)
        v = jnp.dot(x2, wv_ref[...], preferred_element_type=jnp.float32)

        row = lax.broadcasted_iota(jnp.int32, (ROWS, DQK), 0)
        col = lax.broadcasted_iota(jnp.int32, (ROWS, DQK), 1)
        pos = (lax.rem(row, SQ) + my * SQ).astype(jnp.float32)
        expo = (((lax.rem(col, DH) // 2) * 2).astype(jnp.float32)) / DH
        inv = jnp.exp(-expo * math.log(10000.0))
        angle = pos * inv
        cosv = jnp.cos(angle)
        sinv = jnp.sin(angle)

        jj = lax.broadcasted_iota(jnp.int32, (DQK, DQK), 0)
        cc = lax.broadcasted_iota(jnp.int32, (DQK, DQK), 1)
        rot = jnp.where((lax.rem(cc, 2) == 0) & (jj == cc + 1), -1.0,
                        jnp.where((lax.rem(cc, 2) == 1) & (jj == cc - 1),
                                  1.0, 0.0)).astype(jnp.float32)

        q = q * cosv + jnp.dot(q, rot, preferred_element_type=jnp.float32) * sinv
        k = k * cosv + jnp.dot(k, rot, preferred_element_type=jnp.float32) * sinv

        kv_ref[0] = jnp.concatenate([k, v], axis=1).astype(jnp.bfloat16)

        barrier = pltpu.get_barrier_semaphore()
        for nbr in (left, right):
            pl.semaphore_signal(barrier, inc=1, device_id=(nbr,),
                                device_id_type=pl.DeviceIdType.MESH)
        pl.semaphore_wait(barrier, 2)

        lsum = [jnp.zeros((SQ, 1), jnp.float32) for _ in range(B * HQ)]
        acc = [jnp.zeros((SQ, DH), jnp.float32) for _ in range(B * HQ)]

        q_bf = (q * 0.125).astype(jnp.bfloat16)

        def process_b(slot, b):
            r0 = b * SQ
            for hh in range(HQ):
                c0 = hh * DH
                i = b * HQ + hh
                qbh = q_bf[r0:r0 + SQ, c0:c0 + DH]
                kc = kv_ref[slot, r0:r0 + SQ, c0:c0 + DH]
                vc = kv_ref[slot, r0:r0 + SQ, DQK + c0:DQK + c0 + DH]
                s = lax.dot_general(
                    qbh, kc, (((1,), (1,)), ((), ())),
                    preferred_element_type=jnp.float32)
                p = jnp.exp(s)
                lsum[i] = lsum[i] + jnp.sum(p, axis=1, keepdims=True)
                acc[i] = acc[i] + jnp.dot(
                    p.astype(jnp.bfloat16), vc,
                    preferred_element_type=jnp.float32)

        def start(src_slot, dst_slot, sems_s, sems_r, h, half, dst_dev):
            rows = slice(half * SQ, (half + 1) * SQ)
            rdma = pltpu.make_async_remote_copy(
                src_ref=kv_ref.at[src_slot, rows],
                dst_ref=kv_ref.at[dst_slot, rows],
                send_sem=sems_s.at[h, half],
                recv_sem=sems_r.at[h, half],
                device_id=(dst_dev,),
                device_id_type=pl.DeviceIdType.MESH,
            )
            rdma.start()
            return rdma

        dR = [[None, None] for _ in range(R_HOPS)]
        dL = [[None, None] for _ in range(L_HOPS)]

        for half in (0, 1):
            dR[0][half] = start(0, 1, send_r, recv_r, 0, half, right)
            dL[0][half] = start(0, 15, send_l, recv_l, 0, half, left)
        process_b(0, 0)
        process_b(0, 1)

        for h in range(1, R_HOPS + 1):
            for half in (0, 1):
                dR[h - 1][half].wait_recv()
                if h < R_HOPS:
                    dR[h][half] = start(
                        h, h + 1, send_r, recv_r, h, half, right)
                if h <= L_HOPS:
                    dL[h - 1][half].wait_recv()
                    if h < L_HOPS:
                        dL[h][half] = start(
                            16 - h, 15 - h, send_l, recv_l, h, half, left)
            for half in (0, 1):
                process_b(h, half)
                if h <= L_HOPS:
                    process_b(16 - h, half)

        ctx = jnp.concatenate(
            [jnp.concatenate([acc[b * HQ + hh] / lsum[b * HQ + hh]
                              for hh in range(HQ)], axis=1)
             for b in range(B)], axis=0)
        out2 = jnp.dot(ctx, wo_ref[...], preferred_element_type=jnp.float32)
        out_ref[...] = out2.reshape(B, SQ, D)

        for ds in dR + dL:
            for r in ds:
                r.wait_send()

    return pl.pallas_call(
        body,
        out_shape=jax.ShapeDtypeStruct((B, SQ, D), jnp.float32),
        in_specs=[pl.BlockSpec(memory_space=pltpu.VMEM)] * 5,
        out_specs=pl.BlockSpec(memory_space=pltpu.VMEM),
        scratch_shapes=[
            pltpu.VMEM((N_DEV, ROWS, 2 * DQK), jnp.bfloat16),
            pltpu.SemaphoreType.DMA((R_HOPS, 2)),
            pltpu.SemaphoreType.DMA((R_HOPS, 2)),
            pltpu.SemaphoreType.DMA((L_HOPS, 2)),
            pltpu.SemaphoreType.DMA((L_HOPS, 2)),
        ],
        compiler_params=pltpu.CompilerParams(collective_id=0),
    )(x, Wq, Wk, Wv, Wo)


# device time: 38589 ns/iter; 3.3427x vs baseline; 1.0164x over previous
import math

import jax
import jax.numpy as jnp
from jax import lax
from jax.experimental import pallas as pl
from jax.experimental.pallas import tpu as pltpu

N_DEV = 16
B = 2
SQ = 128
D = 512
HQ = 4
DH = 64
DQK = HQ * DH
ROWS = B * SQ
R_HOPS = 8
L_HOPS = 7


def kernel(x, Wq, Wk, Wv, Wo):
    def body(x_ref, wq_ref, wk_ref, wv_ref, wo_ref, out_ref,
             kv_ref, send_r, recv_r, send_l, recv_l):
        my = lax.axis_index("i")

        def ring_pos(m):
            z, c = m // 4, lax.rem(m, 4)
            return c * 4 + jnp.where(lax.rem(c, 2) == 0, z, 3 - z)

        def ring_to_logical(r):
            c, w = r // 4, lax.rem(r, 4)
            z = jnp.where(lax.rem(c, 2) == 0, w, 3 - w)
            return 4 * z + c

        my_r = ring_pos(my)
        right = ring_to_logical(lax.rem(my_r + 1, N_DEV))
        left = ring_to_logical(lax.rem(my_r + N_DEV - 1, N_DEV))

        x2 = x_ref[...].reshape(ROWS, D).astype(jnp.bfloat16)
        q = jnp.dot(x2, wq_ref[...].astype(jnp.bfloat16),
                    preferred_element_type=jnp.float32)
        k = jnp.dot(x2, wk_ref[...].astype(jnp.bfloat16),
                    preferred_element_type=jnp.float32)
        v = jnp.dot(x2, wv_ref[...].astype(jnp.bfloat16),
                    preferred_element_type=jnp.float32)

        row = lax.broadcasted_iota(jnp.int32, (ROWS, DQK), 0)
        col = lax.broadcasted_iota(jnp.int32, (ROWS, DQK), 1)
        pos = (lax.rem(row, SQ) + my * SQ).astype(jnp.float32)
        expo = (((lax.rem(col, DH) // 2) * 2).astype(jnp.float32)) / DH
        inv = jnp.exp(-expo * math.log(10000.0))
        angle = pos * inv
        cosv = jnp.cos(angle)
        sinv = jnp.sin(angle)

        jj = lax.broadcasted_iota(jnp.int32, (DQK, DQK), 0)
        cc = lax.broadcasted_iota(jnp.int32, (DQK, DQK), 1)
        rot = jnp.where((lax.rem(cc, 2) == 0) & (jj == cc + 1), -1.0,
                        jnp.where((lax.rem(cc, 2) == 1) & (jj == cc - 1),
                                  1.0, 0.0)).astype(jnp.bfloat16)

        q = q * cosv + jnp.dot(q.astype(jnp.bfloat16), rot,
                               preferred_element_type=jnp.float32) * sinv
        k = k * cosv + jnp.dot(k.astype(jnp.bfloat16), rot,
                               preferred_element_type=jnp.float32) * sinv

        kv_ref[0] = jnp.concatenate([k, v], axis=1).astype(jnp.bfloat16)

        barrier = pltpu.get_barrier_semaphore()
        for nbr in (left, right):
            pl.semaphore_signal(barrier, inc=1, device_id=(nbr,),
                                device_id_type=pl.DeviceIdType.MESH)
        pl.semaphore_wait(barrier, 2)

        lsum = [jnp.zeros((SQ, 1), jnp.float32) for _ in range(B * HQ)]
        acc = [jnp.zeros((SQ, DH), jnp.float32) for _ in range(B * HQ)]

        q_bf = (q * 0.125).astype(jnp.bfloat16)

        def process_b(slot, b):
            r0 = b * SQ
            for hh in range(HQ):
                c0 = hh * DH
                i = b * HQ + hh
                qbh = q_bf[r0:r0 + SQ, c0:c0 + DH]
                kc = kv_ref[slot, r0:r0 + SQ, c0:c0 + DH]
                vc = kv_ref[slot, r0:r0 + SQ, DQK + c0:DQK + c0 + DH]
                s = lax.dot_general(
                    qbh, kc, (((1,), (1,)), ((), ())),
                    preferred_element_type=jnp.float32)
                p = jnp.exp(s)
                lsum[i] = lsum[i] + jnp.sum(p, axis=1, keepdims=True)
                acc[i] = acc[i] + jnp.dot(
                    p.astype(jnp.bfloat16), vc,
                    preferred_element_type=jnp.float32)

        def start(src_slot, dst_slot, sems_s, sems_r, h, half, dst_dev):
            rows = slice(half * SQ, (half + 1) * SQ)
            rdma = pltpu.make_async_remote_copy(
                src_ref=kv_ref.at[src_slot, rows],
                dst_ref=kv_ref.at[dst_slot, rows],
                send_sem=sems_s.at[h, half],
                recv_sem=sems_r.at[h, half],
                device_id=(dst_dev,),
                device_id_type=pl.DeviceIdType.MESH,
            )
            rdma.start()
            return rdma

        dR = [[None, None] for _ in range(R_HOPS)]
        dL = [[None, None] for _ in range(L_HOPS)]

        for half in (0, 1):
            dR[0][half] = start(0, 1, send_r, recv_r, 0, half, right)
            dL[0][half] = start(0, 15, send_l, recv_l, 0, half, left)
        process_b(0, 0)
        process_b(0, 1)

        for h in range(1, R_HOPS + 1):
            for half in (0, 1):
                dR[h - 1][half].wait_recv()
                if h < R_HOPS:
                    dR[h][half] = start(
                        h, h + 1, send_r, recv_r, h, half, right)
                if h <= L_HOPS:
                    dL[h - 1][half].wait_recv()
                    if h < L_HOPS:
                        dL[h][half] = start(
                            16 - h, 15 - h, send_l, recv_l, h, half, left)
            for half in (0, 1):
                process_b(h, half)
                if h <= L_HOPS:
                    process_b(16 - h, half)

        ctx = jnp.concatenate(
            [jnp.concatenate([acc[b * HQ + hh] / lsum[b * HQ + hh]
                              for hh in range(HQ)], axis=1)
             for b in range(B)], axis=0)
        out2 = jnp.dot(ctx.astype(jnp.bfloat16),
                       wo_ref[...].astype(jnp.bfloat16),
                       preferred_element_type=jnp.float32)
        out_ref[...] = out2.reshape(B, SQ, D)

        for ds in dR + dL:
            for r in ds:
                r.wait_send()

    return pl.pallas_call(
        body,
        out_shape=jax.ShapeDtypeStruct((B, SQ, D), jnp.float32),
        in_specs=[pl.BlockSpec(memory_space=pltpu.VMEM)] * 5,
        out_specs=pl.BlockSpec(memory_space=pltpu.VMEM),
        scratch_shapes=[
            pltpu.VMEM((N_DEV, ROWS, 2 * DQK), jnp.bfloat16),
            pltpu.SemaphoreType.DMA((R_HOPS, 2)),
            pltpu.SemaphoreType.DMA((R_HOPS, 2)),
            pltpu.SemaphoreType.DMA((L_HOPS, 2)),
            pltpu.SemaphoreType.DMA((L_HOPS, 2)),
        ],
        compiler_params=pltpu.CompilerParams(collective_id=0),
    )(x, Wq, Wk, Wv, Wo)
